# Initial kernel scaffold; baseline (speedup 1.0000x reference)
import jax
import jax.numpy as jnp
from jax.experimental import pallas as pl


def kernel(x, y):
    raise NotImplementedError("write your pallas kernel here")


# speedup vs baseline: 57.2170x; 57.2170x over previous
import jax
import jax.numpy as jnp
from jax.experimental import pallas as pl

B, N, D = 4, 4096, 1024
ROWS = 256


def _body(x_ref, xr_ref, y_ref, o0_ref, o1_ref, o2_ref):
    xb = x_ref[...]
    yb = y_ref[...]
    xr = xr_ref[...]

    def sel4(yv, c0, c1, c2, c3):
        return jnp.where(yv == 0, c0,
               jnp.where(yv == 1, c1,
               jnp.where(yv == 2, c2, c3)))

    for b in range(B):
        yv = yb[b]
        o0_ref[b, :, :] = sel4(yv, xb[0], xb[1], xb[2], xb[3])
        o1_ref[b, :, :] = sel4(
            yv,
            jnp.broadcast_to(xr[b, 0][None, :], (ROWS, D)),
            jnp.broadcast_to(xr[b, 1][None, :], (ROWS, D)),
            jnp.broadcast_to(xr[b, 2][None, :], (ROWS, D)),
            jnp.broadcast_to(xr[b, 3][None, :], (ROWS, D)),
        )
        o2_ref[b, :, :] = sel4(
            yv,
            jnp.broadcast_to(xb[b, :, 0][:, None], (ROWS, D)),
            jnp.broadcast_to(xb[b, :, 1][:, None], (ROWS, D)),
            jnp.broadcast_to(xb[b, :, 2][:, None], (ROWS, D)),
            jnp.broadcast_to(xb[b, :, 3][:, None], (ROWS, D)),
        )


def kernel(x, y):
    xr = x[:, 0:4, :]
    grid = (N // ROWS,)
    blk = pl.BlockSpec((B, ROWS, D), lambda i: (0, i, 0))
    out_shape = jax.ShapeDtypeStruct((B, N, D), jnp.float32)
    o0, o1, o2 = pl.pallas_call(
        _body,
        grid=grid,
        in_specs=[
            blk,
            pl.BlockSpec((B, 4, D), lambda i: (0, 0, 0)),
            blk,
        ],
        out_specs=[blk, blk, blk],
        out_shape=[out_shape, out_shape, out_shape],
    )(x, xr, y)
    return (o0, o1, o2)
